# Initial kernel scaffold; baseline (speedup 1.0000x reference)
import jax
import jax.numpy as jnp
from jax.experimental import pallas as pl


def kernel(x, edge_index, enc_W, enc_b, Wq, Wk, Wv, Wo, ln1_g, ln1_b, ffn_W1, ffn_b1, ffn_W2, ffn_b2, ln2_g, ln2_b, dec_W, dec_b):
    raise NotImplementedError("write your pallas kernel here")


# speedup vs baseline: 7.4273x; 7.4273x over previous
import functools

import jax
import jax.numpy as jnp
import numpy as np
from jax import lax
from jax.experimental import pallas as pl
from jax.experimental.pallas import tpu as pltpu
from jax.experimental.pallas import tpu_sc as plsc

N = 10000
E = 320000
D = 128
H = 8
DH = 16
DFF = 512
L = 2

NC = 2
NS = 16
NW = NC * NS
EPS = E // NS
CHUNK = 80
NCHUNK = EPS // CHUNK
ROWW = 144
HALF = 5120
ACCR = HALF + 8
RPT = HALF // NS
SROWS = 8
NPAIR = NCHUNK // 2

_ROWBLK = 1000
_GRID = N // _ROWBLK


def _enc_body(x_ref, w_ref, b_ref, o_ref):
    o_ref[...] = (
        jnp.dot(x_ref[...], w_ref[...], preferred_element_type=jnp.float32)
        + b_ref[...]
    )


def _tc_encode(x, enc_W, enc_b):
    return pl.pallas_call(
        _enc_body,
        grid=(_GRID,),
        in_specs=[
            pl.BlockSpec((_ROWBLK, D), lambda i: (i, 0)),
            pl.BlockSpec((D, D), lambda i: (0, 0)),
            pl.BlockSpec((1, D), lambda i: (0, 0)),
        ],
        out_specs=pl.BlockSpec((_ROWBLK, D), lambda i: (i, 0)),
        out_shape=jax.ShapeDtypeStruct((N, D), jnp.float32),
    )(x, enc_W, enc_b.reshape(1, D))


def _qkv_body(h_ref, wq_ref, wk_ref, wv_ref, q_ref, k_ref, v_ref):
    hb = h_ref[...]
    q_ref[...] = jnp.dot(hb, wq_ref[...], preferred_element_type=jnp.float32)
    k_ref[...] = jnp.dot(hb, wk_ref[...], preferred_element_type=jnp.float32)
    v_ref[...] = jnp.dot(hb, wv_ref[...], preferred_element_type=jnp.float32)


def _tc_qkv(h, Wq, Wk, Wv):
    return pl.pallas_call(
        _qkv_body,
        grid=(_GRID,),
        in_specs=[
            pl.BlockSpec((_ROWBLK, D), lambda i: (i, 0)),
            pl.BlockSpec((D, D), lambda i: (0, 0)),
            pl.BlockSpec((D, D), lambda i: (0, 0)),
            pl.BlockSpec((D, D), lambda i: (0, 0)),
        ],
        out_specs=[
            pl.BlockSpec((_ROWBLK, D), lambda i: (i, 0)),
            pl.BlockSpec((_ROWBLK, D), lambda i: (i, 0)),
            pl.BlockSpec((_ROWBLK, D), lambda i: (i, 0)),
        ],
        out_shape=[jax.ShapeDtypeStruct((N, D), jnp.float32)] * 3,
    )(h, Wq, Wk, Wv)


def _sc_edge_body(qn_hbm, kn_hbm, vn_hbm, eix_hbm, zrows_hbm, out_hbm,
                  eidx, didx2, qb, kb, vb, wrow, acc,
                  sem_i0, sem_i1, sem_q0, sem_k0, sem_v0,
                  sem_q1, sem_k1, sem_v1):
    cid = lax.axis_index("c")
    sid = lax.axis_index("s")
    lane = lax.iota(jnp.int32, 16)
    zeros16 = jnp.zeros((16,), jnp.float32)
    cbase = cid * HALF
    isem = (sem_i0, sem_i1)
    gsem = ((sem_q0, sem_k0, sem_v0), (sem_q1, sem_k1, sem_v1))

    row0 = sid * RPT
    pltpu.sync_copy(zrows_hbm, acc.at[pl.ds(row0, RPT)])

    @pl.when(sid == 0)
    def _():
        pltpu.sync_copy(zrows_hbm.at[pl.ds(0, 8)], acc.at[pl.ds(HALF, 8)])

    plsc.subcore_barrier()

    def issue_idx(j, s):
        pltpu.async_copy(eix_hbm.at[sid].at[j], eidx.at[pl.ds(2 * s, 2)],
                         isem[s])

    def wait_idx(s):
        pltpu.make_async_copy(eix_hbm.at[sid].at[0],
                              eidx.at[pl.ds(2 * s, 2)], isem[s]).wait()

    def issue_gathers(s):
        o = s * CHUNK
        pltpu.async_copy(qn_hbm.at[eidx.at[2 * s + 1]],
                         qb.at[pl.ds(o, CHUNK)], gsem[s][0])
        pltpu.async_copy(kn_hbm.at[eidx.at[2 * s]],
                         kb.at[pl.ds(o, CHUNK)], gsem[s][1])
        pltpu.async_copy(vn_hbm.at[eidx.at[2 * s]],
                         vb.at[pl.ds(o, CHUNK)], gsem[s][2])

    def wait_gathers(s):
        o = s * CHUNK
        for buf, sem in zip((qb, kb, vb), gsem[s]):
            pltpu.make_async_copy(qn_hbm.at[pl.ds(0, CHUNK)],
                                  buf.at[pl.ds(o, CHUNK)], sem).wait()

    def compute(s):
        o = s * CHUNK
        for i in range(CHUNK // 16):
            t = eidx[2 * s + 1, pl.ds(i * 16, 16)] - cbase
            ok = (t >= 0) & (t < HALF)
            didx2[pl.ds(i * 16, 16)] = jnp.where(ok, t, HALF)

        def _edge(e, c2):
            wsum = zeros16
            for h in range(H):
                qv = qb[o + e, pl.ds(h * DH, 16)]
                kv = kb[o + e, pl.ds(h * DH, 16)]
                u = jnp.sum(qv * kv) * 0.25
                u = jnp.clip(u, -5.0, 5.0)
                wv = jnp.exp(jnp.full((16,), u, jnp.float32))
                wrow[e, pl.ds(h * DH, 16)] = wv * vb[o + e, pl.ds(h * DH, 16)]
                wsum = jnp.where(lane == h, wv, wsum)
            wrow[e, pl.ds(D, 16)] = wsum
            return c2

        lax.fori_loop(0, CHUNK, _edge, 0, unroll=4)
        pltpu.sync_copy(wrow, acc.at[didx2], add=True)

    issue_idx(0, 0)
    wait_idx(0)
    issue_gathers(0)
    issue_idx(1, 1)

    def _pair(p, carry):
        wait_gathers(0)
        wait_idx(1)
        issue_gathers(1)

        @pl.when(p < NPAIR - 1)
        def _():
            issue_idx(2 * p + 2, 0)

        compute(0)

        wait_gathers(1)

        @pl.when(p < NPAIR - 1)
        def _():
            wait_idx(0)
            issue_gathers(0)
            issue_idx(2 * p + 3, 1)

        compute(1)
        return carry

    lax.fori_loop(0, NPAIR, _pair, 0)
    plsc.subcore_barrier()

    pltpu.sync_copy(acc.at[pl.ds(row0, RPT)],
                    out_hbm.at[cid].at[pl.ds(row0, RPT)])


@functools.cache
def _sc_edge():
    return pl.kernel(
        _sc_edge_body,
        out_type=jax.ShapeDtypeStruct((NC, HALF, ROWW), jnp.float32),
        mesh=plsc.VectorSubcoreMesh(
            core_axis_name="c", subcore_axis_name="s",
            num_cores=NC, num_subcores=NS),
        compiler_params=pltpu.CompilerParams(
            use_tc_tiling_on_sc=False, needs_layout_passes=False),
        scratch_types=[
            pltpu.VMEM((4, CHUNK), jnp.int32),
            pltpu.VMEM((CHUNK,), jnp.int32),
            pltpu.VMEM((2 * CHUNK, D), jnp.float32),
            pltpu.VMEM((2 * CHUNK, D), jnp.float32),
            pltpu.VMEM((2 * CHUNK, D), jnp.float32),
            pltpu.VMEM((CHUNK, ROWW), jnp.float32),
            pltpu.VMEM_SHARED((ACCR, ROWW), jnp.float32),
        ] + [pltpu.SemaphoreType.DMA] * 8,
    )


def _ln(t, g, b):
    mu = jnp.mean(t, axis=-1, keepdims=True)
    d = t - mu
    var = jnp.mean(d * d, axis=-1, keepdims=True)
    return d / jnp.sqrt(var + 1e-5) * g + b


def _post_body(h_ref, p_ref, wo_ref, l1g_ref, l1b_ref,
               w1_ref, b1_ref, w2_ref, b2_ref, l2g_ref, l2b_ref,
               exp8_ref, o_ref):
    p = p_ref[...]
    num = p[:, :D]
    ssum = p[:, D:D + H]
    inv = 1.0 / (ssum + 1e-30)
    inv128 = jnp.dot(inv, exp8_ref[...], preferred_element_type=jnp.float32)
    agg = num * inv128
    uh = jnp.dot(agg, wo_ref[...], preferred_element_type=jnp.float32)
    h1 = _ln(h_ref[...] + uh, l1g_ref[...], l1b_ref[...])
    z = jnp.maximum(
        jnp.dot(h1, w1_ref[...], preferred_element_type=jnp.float32)
        + b1_ref[...], 0.0)
    ffn = jnp.dot(z, w2_ref[...], preferred_element_type=jnp.float32) + b2_ref[...]
    o_ref[...] = _ln(h1 + ffn, l2g_ref[...], l2b_ref[...])


def _tc_post(h, part, Wo_l, l1g, l1b, W1, b1, W2, b2, l2g, l2b, exp8):
    return pl.pallas_call(
        _post_body,
        grid=(_GRID,),
        in_specs=[
            pl.BlockSpec((_ROWBLK, D), lambda i: (i, 0)),
            pl.BlockSpec((_ROWBLK, ROWW), lambda i: (i, 0)),
            pl.BlockSpec((D, D), lambda i: (0, 0)),
            pl.BlockSpec((1, D), lambda i: (0, 0)),
            pl.BlockSpec((1, D), lambda i: (0, 0)),
            pl.BlockSpec((D, DFF), lambda i: (0, 0)),
            pl.BlockSpec((1, DFF), lambda i: (0, 0)),
            pl.BlockSpec((DFF, D), lambda i: (0, 0)),
            pl.BlockSpec((1, D), lambda i: (0, 0)),
            pl.BlockSpec((1, D), lambda i: (0, 0)),
            pl.BlockSpec((1, D), lambda i: (0, 0)),
            pl.BlockSpec((H, D), lambda i: (0, 0)),
        ],
        out_specs=pl.BlockSpec((_ROWBLK, D), lambda i: (i, 0)),
        out_shape=jax.ShapeDtypeStruct((N, D), jnp.float32),
    )(h, part, Wo_l, l1g.reshape(1, D), l1b.reshape(1, D),
      W1, b1.reshape(1, DFF), W2, b2.reshape(1, D),
      l2g.reshape(1, D), l2b.reshape(1, D), exp8)


def _dec_body(h_ref, wt_ref, b_ref, o_ref):
    i = pl.program_id(0)
    s = jnp.sum(h_ref[...] * wt_ref[...]) * (1.0 / N)

    @pl.when(i == 0)
    def _():
        o_ref[...] = jnp.zeros_like(o_ref)

    o_ref[...] += s

    @pl.when(i == _GRID - 1)
    def _():
        o_ref[...] += b_ref[...]


def _tc_decode(h, dec_W, dec_b):
    return pl.pallas_call(
        _dec_body,
        grid=(_GRID,),
        in_specs=[
            pl.BlockSpec((_ROWBLK, D), lambda i: (i, 0)),
            pl.BlockSpec((1, D), lambda i: (0, 0)),
            pl.BlockSpec((1, 1), lambda i: (0, 0)),
        ],
        out_specs=pl.BlockSpec((1, 1), lambda i: (0, 0)),
        out_shape=jax.ShapeDtypeStruct((1, 1), jnp.float32),
    )(h, dec_W.reshape(1, D), dec_b.reshape(1, 1))


_EXP8 = np.repeat(np.eye(H, dtype=np.float32), DH, axis=1)


def kernel(x, edge_index, enc_W, enc_b, Wq, Wk, Wv, Wo, ln1_g, ln1_b,
           ffn_W1, ffn_b1, ffn_W2, ffn_b2, ln2_g, ln2_b, dec_W, dec_b):
    src = edge_index[0].reshape(NS, NCHUNK, CHUNK)
    dst = edge_index[1].reshape(NS, NCHUNK, CHUNK)
    eix = jnp.stack([src, dst], axis=2)
    zrows = jnp.zeros((RPT, ROWW), jnp.float32)
    exp8 = jnp.asarray(_EXP8)

    h = _tc_encode(x, enc_W, enc_b)
    for l in range(L):
        qn, kn, vn = _tc_qkv(h, Wq[l], Wk[l], Wv[l])
        part = _sc_edge()(qn, kn, vn, eix, zrows).reshape(NC * HALF, ROWW)
        h = _tc_post(h, part, Wo[l], ln1_g[l], ln1_b[l],
                     ffn_W1[l], ffn_b1[l], ffn_W2[l], ffn_b2[l],
                     ln2_g[l], ln2_b[l], exp8)
    return _tc_decode(h, dec_W, dec_b)


# speedup vs baseline: 10.6789x; 1.4378x over previous
import functools

import jax
import jax.numpy as jnp
import numpy as np
from jax import lax
from jax.experimental import pallas as pl
from jax.experimental.pallas import tpu as pltpu
from jax.experimental.pallas import tpu_sc as plsc

N = 10000
E = 320000
D = 128
H = 8
DH = 16
DFF = 512
L = 2

NC = 2
NS = 16
NW = NC * NS
EPS = E // NS
CHUNK = 80
NCHUNK = EPS // CHUNK
ROWW = 144
HALF = 5120
ACCR = HALF + 8
RPT = HALF // NS
SROWS = 8
NPAIR = NCHUNK // 2

_ROWBLK = 1000
_GRID = N // _ROWBLK


def _enc_body(x_ref, w_ref, b_ref, o_ref):
    o_ref[...] = (
        jnp.dot(x_ref[...], w_ref[...], preferred_element_type=jnp.float32)
        + b_ref[...]
    )


def _tc_encode(x, enc_W, enc_b):
    return pl.pallas_call(
        _enc_body,
        grid=(_GRID,),
        in_specs=[
            pl.BlockSpec((_ROWBLK, D), lambda i: (i, 0)),
            pl.BlockSpec((D, D), lambda i: (0, 0)),
            pl.BlockSpec((1, D), lambda i: (0, 0)),
        ],
        out_specs=pl.BlockSpec((_ROWBLK, D), lambda i: (i, 0)),
        out_shape=jax.ShapeDtypeStruct((N, D), jnp.float32),
    )(x, enc_W, enc_b.reshape(1, D))


def _qkv_body(h_ref, wq_ref, wk_ref, wv_ref, q_ref, k_ref, v_ref):
    hb = h_ref[...]
    q_ref[...] = jnp.dot(hb, wq_ref[...], preferred_element_type=jnp.float32)
    k_ref[...] = jnp.dot(hb, wk_ref[...], preferred_element_type=jnp.float32)
    v_ref[...] = jnp.dot(hb, wv_ref[...], preferred_element_type=jnp.float32)


def _tc_qkv(h, Wq, Wk, Wv):
    return pl.pallas_call(
        _qkv_body,
        grid=(_GRID,),
        in_specs=[
            pl.BlockSpec((_ROWBLK, D), lambda i: (i, 0)),
            pl.BlockSpec((D, D), lambda i: (0, 0)),
            pl.BlockSpec((D, D), lambda i: (0, 0)),
            pl.BlockSpec((D, D), lambda i: (0, 0)),
        ],
        out_specs=[
            pl.BlockSpec((_ROWBLK, D), lambda i: (i, 0)),
            pl.BlockSpec((_ROWBLK, D), lambda i: (i, 0)),
            pl.BlockSpec((_ROWBLK, D), lambda i: (i, 0)),
        ],
        out_shape=[jax.ShapeDtypeStruct((N, D), jnp.float32)] * 3,
    )(h, Wq, Wk, Wv)


def _splat(x, h):
    idx = jnp.full((16, 1), h, jnp.int32)
    return lax.gather(
        x, idx,
        lax.GatherDimensionNumbers(offset_dims=(), collapsed_slice_dims=(0,),
                                   start_index_map=(0,)),
        (1,), mode=lax.GatherScatterMode.PROMISE_IN_BOUNDS)


def _sc_edge_body(qn_hbm, kn_hbm, vn_hbm, eix_hbm, zrows_hbm, out_hbm,
                  eidx, didx2, qb, kb, vb, wrow, wtmp, acc,
                  sem_i0, sem_i1, sem_q0, sem_k0, sem_v0,
                  sem_q1, sem_k1, sem_v1):
    cid = lax.axis_index("c")
    sid = lax.axis_index("s")
    lane = lax.iota(jnp.int32, 16)
    zeros16 = jnp.zeros((16,), jnp.float32)
    cbase = cid * HALF
    isem = (sem_i0, sem_i1)
    gsem = ((sem_q0, sem_k0, sem_v0), (sem_q1, sem_k1, sem_v1))

    row0 = sid * RPT
    pltpu.sync_copy(zrows_hbm, acc.at[pl.ds(row0, RPT)])

    @pl.when(sid == 0)
    def _():
        pltpu.sync_copy(zrows_hbm.at[pl.ds(0, 8)], acc.at[pl.ds(HALF, 8)])

    plsc.subcore_barrier()

    def issue_idx(j, s):
        pltpu.async_copy(eix_hbm.at[sid].at[j], eidx.at[pl.ds(2 * s, 2)],
                         isem[s])

    def wait_idx(s):
        pltpu.make_async_copy(eix_hbm.at[sid].at[0],
                              eidx.at[pl.ds(2 * s, 2)], isem[s]).wait()

    def issue_gathers(s):
        o = s * CHUNK
        pltpu.async_copy(qn_hbm.at[eidx.at[2 * s + 1]],
                         qb.at[pl.ds(o, CHUNK)], gsem[s][0])
        pltpu.async_copy(kn_hbm.at[eidx.at[2 * s]],
                         kb.at[pl.ds(o, CHUNK)], gsem[s][1])
        pltpu.async_copy(vn_hbm.at[eidx.at[2 * s]],
                         vb.at[pl.ds(o, CHUNK)], gsem[s][2])

    def wait_gathers(s):
        o = s * CHUNK
        for buf, sem in zip((qb, kb, vb), gsem[s]):
            pltpu.make_async_copy(qn_hbm.at[pl.ds(0, CHUNK)],
                                  buf.at[pl.ds(o, CHUNK)], sem).wait()

    def compute(s):
        o = s * CHUNK
        for i in range(CHUNK // 16):
            t = eidx[2 * s + 1, pl.ds(i * 16, 16)] - cbase
            ok = (t >= 0) & (t < HALF)
            didx2[pl.ds(i * 16, 16)] = jnp.where(ok, t, HALF)

        def _group(g, c2):
            base = o + g * DH
            rows = lane + base
            for h in range(H):
                ua = zeros16
                for c in range(DH):
                    col = jnp.full((16,), h * DH + c, jnp.int32)
                    qc = plsc.load_gather(qb, [rows, col])
                    kc = plsc.load_gather(kb, [rows, col])
                    ua = ua + qc * kc
                u = jnp.clip(ua * 0.25, -5.0, 5.0)
                wtmp[h, :] = jnp.exp(u)
            for e in range(DH):
                ec = g * DH + e
                wvec = plsc.load_gather(wtmp, [lane, jnp.full((16,), e, jnp.int32)])
                wrow[ec, pl.ds(D, 16)] = wvec
                for h in range(H):
                    ws = _splat(wvec, h)
                    wrow[ec, pl.ds(h * DH, 16)] = ws * vb[base + e, pl.ds(h * DH, 16)]
            return c2

        lax.fori_loop(0, CHUNK // 16, _group, 0)
        pltpu.sync_copy(wrow, acc.at[didx2], add=True)

    issue_idx(0, 0)
    wait_idx(0)
    issue_gathers(0)
    issue_idx(1, 1)

    def _pair(p, carry):
        wait_gathers(0)
        wait_idx(1)
        issue_gathers(1)

        @pl.when(p < NPAIR - 1)
        def _():
            issue_idx(2 * p + 2, 0)

        compute(0)

        wait_gathers(1)

        @pl.when(p < NPAIR - 1)
        def _():
            wait_idx(0)
            issue_gathers(0)
            issue_idx(2 * p + 3, 1)

        compute(1)
        return carry

    lax.fori_loop(0, NPAIR, _pair, 0)
    plsc.subcore_barrier()

    pltpu.sync_copy(acc.at[pl.ds(row0, RPT)],
                    out_hbm.at[cid].at[pl.ds(row0, RPT)])


@functools.cache
def _sc_edge():
    return pl.kernel(
        _sc_edge_body,
        out_type=jax.ShapeDtypeStruct((NC, HALF, ROWW), jnp.float32),
        mesh=plsc.VectorSubcoreMesh(
            core_axis_name="c", subcore_axis_name="s",
            num_cores=NC, num_subcores=NS),
        compiler_params=pltpu.CompilerParams(
            use_tc_tiling_on_sc=False, needs_layout_passes=False),
        scratch_types=[
            pltpu.VMEM((4, CHUNK), jnp.int32),
            pltpu.VMEM((CHUNK,), jnp.int32),
            pltpu.VMEM((2 * CHUNK, D), jnp.float32),
            pltpu.VMEM((2 * CHUNK, D), jnp.float32),
            pltpu.VMEM((2 * CHUNK, D), jnp.float32),
            pltpu.VMEM((CHUNK, ROWW), jnp.float32),
            pltpu.VMEM((16, 16), jnp.float32),
            pltpu.VMEM_SHARED((ACCR, ROWW), jnp.float32),
        ] + [pltpu.SemaphoreType.DMA] * 8,
    )


def _ln(t, g, b):
    mu = jnp.mean(t, axis=-1, keepdims=True)
    d = t - mu
    var = jnp.mean(d * d, axis=-1, keepdims=True)
    return d / jnp.sqrt(var + 1e-5) * g + b


def _post_body(h_ref, p_ref, wo_ref, l1g_ref, l1b_ref,
               w1_ref, b1_ref, w2_ref, b2_ref, l2g_ref, l2b_ref,
               exp8_ref, o_ref):
    p = p_ref[...]
    num = p[:, :D]
    ssum = p[:, D:D + H]
    inv = 1.0 / (ssum + 1e-30)
    inv128 = jnp.dot(inv, exp8_ref[...], preferred_element_type=jnp.float32)
    agg = num * inv128
    uh = jnp.dot(agg, wo_ref[...], preferred_element_type=jnp.float32)
    h1 = _ln(h_ref[...] + uh, l1g_ref[...], l1b_ref[...])
    z = jnp.maximum(
        jnp.dot(h1, w1_ref[...], preferred_element_type=jnp.float32)
        + b1_ref[...], 0.0)
    ffn = jnp.dot(z, w2_ref[...], preferred_element_type=jnp.float32) + b2_ref[...]
    o_ref[...] = _ln(h1 + ffn, l2g_ref[...], l2b_ref[...])


def _tc_post(h, part, Wo_l, l1g, l1b, W1, b1, W2, b2, l2g, l2b, exp8):
    return pl.pallas_call(
        _post_body,
        grid=(_GRID,),
        in_specs=[
            pl.BlockSpec((_ROWBLK, D), lambda i: (i, 0)),
            pl.BlockSpec((_ROWBLK, ROWW), lambda i: (i, 0)),
            pl.BlockSpec((D, D), lambda i: (0, 0)),
            pl.BlockSpec((1, D), lambda i: (0, 0)),
            pl.BlockSpec((1, D), lambda i: (0, 0)),
            pl.BlockSpec((D, DFF), lambda i: (0, 0)),
            pl.BlockSpec((1, DFF), lambda i: (0, 0)),
            pl.BlockSpec((DFF, D), lambda i: (0, 0)),
            pl.BlockSpec((1, D), lambda i: (0, 0)),
            pl.BlockSpec((1, D), lambda i: (0, 0)),
            pl.BlockSpec((1, D), lambda i: (0, 0)),
            pl.BlockSpec((H, D), lambda i: (0, 0)),
        ],
        out_specs=pl.BlockSpec((_ROWBLK, D), lambda i: (i, 0)),
        out_shape=jax.ShapeDtypeStruct((N, D), jnp.float32),
    )(h, part, Wo_l, l1g.reshape(1, D), l1b.reshape(1, D),
      W1, b1.reshape(1, DFF), W2, b2.reshape(1, D),
      l2g.reshape(1, D), l2b.reshape(1, D), exp8)


def _dec_body(h_ref, wt_ref, b_ref, o_ref):
    i = pl.program_id(0)
    s = jnp.sum(h_ref[...] * wt_ref[...]) * (1.0 / N)

    @pl.when(i == 0)
    def _():
        o_ref[...] = jnp.zeros_like(o_ref)

    o_ref[...] += s

    @pl.when(i == _GRID - 1)
    def _():
        o_ref[...] += b_ref[...]


def _tc_decode(h, dec_W, dec_b):
    return pl.pallas_call(
        _dec_body,
        grid=(_GRID,),
        in_specs=[
            pl.BlockSpec((_ROWBLK, D), lambda i: (i, 0)),
            pl.BlockSpec((1, D), lambda i: (0, 0)),
            pl.BlockSpec((1, 1), lambda i: (0, 0)),
        ],
        out_specs=pl.BlockSpec((1, 1), lambda i: (0, 0)),
        out_shape=jax.ShapeDtypeStruct((1, 1), jnp.float32),
    )(h, dec_W.reshape(1, D), dec_b.reshape(1, 1))


_EXP8 = np.repeat(np.eye(H, dtype=np.float32), DH, axis=1)


def kernel(x, edge_index, enc_W, enc_b, Wq, Wk, Wv, Wo, ln1_g, ln1_b,
           ffn_W1, ffn_b1, ffn_W2, ffn_b2, ln2_g, ln2_b, dec_W, dec_b):
    src = edge_index[0].reshape(NS, NCHUNK, CHUNK)
    dst = edge_index[1].reshape(NS, NCHUNK, CHUNK)
    eix = jnp.stack([src, dst], axis=2)
    zrows = jnp.zeros((RPT, ROWW), jnp.float32)
    exp8 = jnp.asarray(_EXP8)

    h = _tc_encode(x, enc_W, enc_b)
    for l in range(L):
        qn, kn, vn = _tc_qkv(h, Wq[l], Wk[l], Wv[l])
        part = _sc_edge()(qn, kn, vn, eix, zrows).reshape(NC * HALF, ROWW)
        h = _tc_post(h, part, Wo[l], ln1_g[l], ln1_b[l],
                     ffn_W1[l], ffn_b1[l], ffn_W2[l], ffn_b2[l],
                     ln2_g[l], ln2_b[l], exp8)
    return _tc_decode(h, dec_W, dec_b)


# speedup vs baseline: 11.0051x; 1.0305x over previous
import functools

import jax
import jax.numpy as jnp
import numpy as np
from jax import lax
from jax.experimental import pallas as pl
from jax.experimental.pallas import tpu as pltpu
from jax.experimental.pallas import tpu_sc as plsc

N = 10000
E = 320000
D = 128
H = 8
DH = 16
DFF = 512
L = 2

NC = 2
NS = 16
NW = NC * NS
EPS = E // NS
CHUNK = 80
NCHUNK = EPS // CHUNK
ROWW = 144
HALF = 5120
ACCR = HALF + 8
RPT = HALF // NS
SROWS = 8
NPAIR = NCHUNK // 2

_ROWBLK = 1000
_GRID = N // _ROWBLK


def _enc_body(x_ref, w_ref, b_ref, o_ref):
    o_ref[...] = (
        jnp.dot(x_ref[...], w_ref[...], preferred_element_type=jnp.float32)
        + b_ref[...]
    )


def _tc_encode(x, enc_W, enc_b):
    return pl.pallas_call(
        _enc_body,
        grid=(_GRID,),
        in_specs=[
            pl.BlockSpec((_ROWBLK, D), lambda i: (i, 0)),
            pl.BlockSpec((D, D), lambda i: (0, 0)),
            pl.BlockSpec((1, D), lambda i: (0, 0)),
        ],
        out_specs=pl.BlockSpec((_ROWBLK, D), lambda i: (i, 0)),
        out_shape=jax.ShapeDtypeStruct((N, D), jnp.float32),
    )(x, enc_W, enc_b.reshape(1, D))


def _qkv_body(h_ref, wq_ref, wk_ref, wv_ref, q_ref, k_ref, v_ref):
    hb = h_ref[...]
    q_ref[...] = jnp.dot(hb, wq_ref[...], preferred_element_type=jnp.float32)
    k_ref[...] = jnp.dot(hb, wk_ref[...], preferred_element_type=jnp.float32)
    v_ref[...] = jnp.dot(hb, wv_ref[...], preferred_element_type=jnp.float32)


def _tc_qkv(h, Wq, Wk, Wv):
    return pl.pallas_call(
        _qkv_body,
        grid=(_GRID,),
        in_specs=[
            pl.BlockSpec((_ROWBLK, D), lambda i: (i, 0)),
            pl.BlockSpec((D, D), lambda i: (0, 0)),
            pl.BlockSpec((D, D), lambda i: (0, 0)),
            pl.BlockSpec((D, D), lambda i: (0, 0)),
        ],
        out_specs=[
            pl.BlockSpec((_ROWBLK, D), lambda i: (i, 0)),
            pl.BlockSpec((_ROWBLK, D), lambda i: (i, 0)),
            pl.BlockSpec((_ROWBLK, D), lambda i: (i, 0)),
        ],
        out_shape=[jax.ShapeDtypeStruct((N, D), jnp.float32)] * 3,
    )(h, Wq, Wk, Wv)


def _splat(x, h):
    idx = jnp.full((16, 1), h, jnp.int32)
    return lax.gather(
        x, idx,
        lax.GatherDimensionNumbers(offset_dims=(), collapsed_slice_dims=(0,),
                                   start_index_map=(0,)),
        (1,), mode=lax.GatherScatterMode.PROMISE_IN_BOUNDS)


def _sc_edge_body(qn_hbm, kn_hbm, vn_hbm, eix_hbm, zrows_hbm, out_hbm,
                  eidx, didx2, qb, kb, vb, wrow, wtmp, acc,
                  sem_i0, sem_i1, sem_q0, sem_k0, sem_v0,
                  sem_q1, sem_k1, sem_v1):
    cid = lax.axis_index("c")
    sid = lax.axis_index("s")
    lane = lax.iota(jnp.int32, 16)
    zeros16 = jnp.zeros((16,), jnp.float32)
    cbase = cid * HALF
    isem = (sem_i0, sem_i1)
    gsem = ((sem_q0, sem_k0, sem_v0), (sem_q1, sem_k1, sem_v1))

    row0 = sid * RPT
    pltpu.sync_copy(zrows_hbm, acc.at[pl.ds(row0, RPT)])

    @pl.when(sid == 0)
    def _():
        pltpu.sync_copy(zrows_hbm.at[pl.ds(0, 8)], acc.at[pl.ds(HALF, 8)])

    plsc.subcore_barrier()

    def issue_idx(j, s):
        pltpu.async_copy(eix_hbm.at[sid].at[j], eidx.at[pl.ds(2 * s, 2)],
                         isem[s])

    def wait_idx(s):
        pltpu.make_async_copy(eix_hbm.at[sid].at[0],
                              eidx.at[pl.ds(2 * s, 2)], isem[s]).wait()

    def issue_gathers(s):
        o = s * CHUNK
        pltpu.async_copy(qn_hbm.at[eidx.at[2 * s + 1]],
                         qb.at[pl.ds(o, CHUNK)], gsem[s][0])
        pltpu.async_copy(kn_hbm.at[eidx.at[2 * s]],
                         kb.at[pl.ds(o, CHUNK)], gsem[s][1])
        pltpu.async_copy(vn_hbm.at[eidx.at[2 * s]],
                         vb.at[pl.ds(o, CHUNK)], gsem[s][2])

    def wait_gathers(s):
        o = s * CHUNK
        for buf, sem in zip((qb, kb, vb), gsem[s]):
            pltpu.make_async_copy(qn_hbm.at[pl.ds(0, CHUNK)],
                                  buf.at[pl.ds(o, CHUNK)], sem).wait()

    def compute(s):
        o = s * CHUNK
        for i in range(CHUNK // 16):
            t = eidx[2 * s + 1, pl.ds(i * 16, 16)] - cbase
            ok = (t >= 0) & (t < HALF)
            didx2[pl.ds(i * 16, 16)] = jnp.where(ok, t, HALF)

        def _group(g, c2):
            base = o + g * DH
            rows = lane + base
            ua = [zeros16] * H
            for c in range(DH):
                for h in range(H):
                    col = jnp.full((16,), h * DH + c, jnp.int32)
                    qc = plsc.load_gather(qb, [rows, col])
                    kc = plsc.load_gather(kb, [rows, col])
                    ua[h] = ua[h] + qc * kc
            for h in range(H):
                u = jnp.clip(ua[h] * 0.25, -5.0, 5.0)
                wtmp[h, :] = jnp.exp(u)
            for e in range(DH):
                ec = g * DH + e
                wvec = plsc.load_gather(wtmp, [lane, jnp.full((16,), e, jnp.int32)])
                wrow[ec, pl.ds(D, 16)] = wvec
                for h in range(H):
                    ws = _splat(wvec, h)
                    wrow[ec, pl.ds(h * DH, 16)] = ws * vb[base + e, pl.ds(h * DH, 16)]
            return c2

        lax.fori_loop(0, CHUNK // 16, _group, 0)
        pltpu.sync_copy(wrow, acc.at[didx2], add=True)

    issue_idx(0, 0)
    wait_idx(0)
    issue_gathers(0)
    issue_idx(1, 1)

    def _pair(p, carry):
        wait_gathers(0)
        wait_idx(1)
        issue_gathers(1)

        @pl.when(p < NPAIR - 1)
        def _():
            issue_idx(2 * p + 2, 0)

        compute(0)

        wait_gathers(1)

        @pl.when(p < NPAIR - 1)
        def _():
            wait_idx(0)
            issue_gathers(0)
            issue_idx(2 * p + 3, 1)

        compute(1)
        return carry

    lax.fori_loop(0, NPAIR, _pair, 0)
    plsc.subcore_barrier()

    pltpu.sync_copy(acc.at[pl.ds(row0, RPT)],
                    out_hbm.at[cid].at[pl.ds(row0, RPT)])


@functools.cache
def _sc_edge():
    return pl.kernel(
        _sc_edge_body,
        out_type=jax.ShapeDtypeStruct((NC, HALF, ROWW), jnp.float32),
        mesh=plsc.VectorSubcoreMesh(
            core_axis_name="c", subcore_axis_name="s",
            num_cores=NC, num_subcores=NS),
        compiler_params=pltpu.CompilerParams(
            use_tc_tiling_on_sc=False, needs_layout_passes=False),
        scratch_types=[
            pltpu.VMEM((4, CHUNK), jnp.int32),
            pltpu.VMEM((CHUNK,), jnp.int32),
            pltpu.VMEM((2 * CHUNK, D), jnp.float32),
            pltpu.VMEM((2 * CHUNK, D), jnp.float32),
            pltpu.VMEM((2 * CHUNK, D), jnp.float32),
            pltpu.VMEM((CHUNK, ROWW), jnp.float32),
            pltpu.VMEM((16, 16), jnp.float32),
            pltpu.VMEM_SHARED((ACCR, ROWW), jnp.float32),
        ] + [pltpu.SemaphoreType.DMA] * 8,
    )


def _ln(t, g, b):
    mu = jnp.mean(t, axis=-1, keepdims=True)
    d = t - mu
    var = jnp.mean(d * d, axis=-1, keepdims=True)
    return d / jnp.sqrt(var + 1e-5) * g + b


def _post_body(h_ref, p_ref, wo_ref, l1g_ref, l1b_ref,
               w1_ref, b1_ref, w2_ref, b2_ref, l2g_ref, l2b_ref,
               exp8_ref, o_ref):
    p = p_ref[...]
    num = p[:, :D]
    ssum = p[:, D:D + H]
    inv = 1.0 / (ssum + 1e-30)
    inv128 = jnp.dot(inv, exp8_ref[...], preferred_element_type=jnp.float32)
    agg = num * inv128
    uh = jnp.dot(agg, wo_ref[...], preferred_element_type=jnp.float32)
    h1 = _ln(h_ref[...] + uh, l1g_ref[...], l1b_ref[...])
    z = jnp.maximum(
        jnp.dot(h1, w1_ref[...], preferred_element_type=jnp.float32)
        + b1_ref[...], 0.0)
    ffn = jnp.dot(z, w2_ref[...], preferred_element_type=jnp.float32) + b2_ref[...]
    o_ref[...] = _ln(h1 + ffn, l2g_ref[...], l2b_ref[...])


def _tc_post(h, part, Wo_l, l1g, l1b, W1, b1, W2, b2, l2g, l2b, exp8):
    return pl.pallas_call(
        _post_body,
        grid=(_GRID,),
        in_specs=[
            pl.BlockSpec((_ROWBLK, D), lambda i: (i, 0)),
            pl.BlockSpec((_ROWBLK, ROWW), lambda i: (i, 0)),
            pl.BlockSpec((D, D), lambda i: (0, 0)),
            pl.BlockSpec((1, D), lambda i: (0, 0)),
            pl.BlockSpec((1, D), lambda i: (0, 0)),
            pl.BlockSpec((D, DFF), lambda i: (0, 0)),
            pl.BlockSpec((1, DFF), lambda i: (0, 0)),
            pl.BlockSpec((DFF, D), lambda i: (0, 0)),
            pl.BlockSpec((1, D), lambda i: (0, 0)),
            pl.BlockSpec((1, D), lambda i: (0, 0)),
            pl.BlockSpec((1, D), lambda i: (0, 0)),
            pl.BlockSpec((H, D), lambda i: (0, 0)),
        ],
        out_specs=pl.BlockSpec((_ROWBLK, D), lambda i: (i, 0)),
        out_shape=jax.ShapeDtypeStruct((N, D), jnp.float32),
    )(h, part, Wo_l, l1g.reshape(1, D), l1b.reshape(1, D),
      W1, b1.reshape(1, DFF), W2, b2.reshape(1, D),
      l2g.reshape(1, D), l2b.reshape(1, D), exp8)


def _dec_body(h_ref, wt_ref, b_ref, o_ref):
    i = pl.program_id(0)
    s = jnp.sum(h_ref[...] * wt_ref[...]) * (1.0 / N)

    @pl.when(i == 0)
    def _():
        o_ref[...] = jnp.zeros_like(o_ref)

    o_ref[...] += s

    @pl.when(i == _GRID - 1)
    def _():
        o_ref[...] += b_ref[...]


def _tc_decode(h, dec_W, dec_b):
    return pl.pallas_call(
        _dec_body,
        grid=(_GRID,),
        in_specs=[
            pl.BlockSpec((_ROWBLK, D), lambda i: (i, 0)),
            pl.BlockSpec((1, D), lambda i: (0, 0)),
            pl.BlockSpec((1, 1), lambda i: (0, 0)),
        ],
        out_specs=pl.BlockSpec((1, 1), lambda i: (0, 0)),
        out_shape=jax.ShapeDtypeStruct((1, 1), jnp.float32),
    )(h, dec_W.reshape(1, D), dec_b.reshape(1, 1))


_EXP8 = np.repeat(np.eye(H, dtype=np.float32), DH, axis=1)


def kernel(x, edge_index, enc_W, enc_b, Wq, Wk, Wv, Wo, ln1_g, ln1_b,
           ffn_W1, ffn_b1, ffn_W2, ffn_b2, ln2_g, ln2_b, dec_W, dec_b):
    src = edge_index[0].reshape(NS, NCHUNK, CHUNK)
    dst = edge_index[1].reshape(NS, NCHUNK, CHUNK)
    eix = jnp.stack([src, dst], axis=2)
    zrows = jnp.zeros((RPT, ROWW), jnp.float32)
    exp8 = jnp.asarray(_EXP8)

    h = _tc_encode(x, enc_W, enc_b)
    for l in range(L):
        qn, kn, vn = _tc_qkv(h, Wq[l], Wk[l], Wv[l])
        part = _sc_edge()(qn, kn, vn, eix, zrows).reshape(NC * HALF, ROWW)
        h = _tc_post(h, part, Wo[l], ln1_g[l], ln1_b[l],
                     ffn_W1[l], ffn_b1[l], ffn_W2[l], ffn_b2[l],
                     ln2_g[l], ln2_b[l], exp8)
    return _tc_decode(h, dec_W, dec_b)


# speedup vs baseline: 18.7988x; 1.7082x over previous
import functools

import jax
import jax.numpy as jnp
import numpy as np
from jax import lax
from jax.experimental import pallas as pl
from jax.experimental.pallas import tpu as pltpu
from jax.experimental.pallas import tpu_sc as plsc

N = 10000
E = 320000
D = 128
H = 8
DH = 16
DFF = 512
L = 2

NC = 2
NS = 16
NW = NC * NS
EPS = E // NS
CHUNK = 80
NCHUNK = EPS // CHUNK
ROWW = 144
HALF = 5120
ACCR = HALF + 8
RPT = HALF // NS
SROWS = 8
NPAIR = NCHUNK // 2

_ROWBLK = 1000
_GRID = N // _ROWBLK


def _enc_body(x_ref, w_ref, b_ref, o_ref):
    o_ref[...] = (
        jnp.dot(x_ref[...], w_ref[...], preferred_element_type=jnp.float32)
        + b_ref[...]
    )


def _tc_encode(x, enc_W, enc_b):
    return pl.pallas_call(
        _enc_body,
        grid=(_GRID,),
        in_specs=[
            pl.BlockSpec((_ROWBLK, D), lambda i: (i, 0)),
            pl.BlockSpec((D, D), lambda i: (0, 0)),
            pl.BlockSpec((1, D), lambda i: (0, 0)),
        ],
        out_specs=pl.BlockSpec((_ROWBLK, D), lambda i: (i, 0)),
        out_shape=jax.ShapeDtypeStruct((N, D), jnp.float32),
    )(x, enc_W, enc_b.reshape(1, D))


def _qkv_body(h_ref, wq_ref, wk_ref, wv_ref, q_ref, k_ref, v_ref):
    hb = h_ref[...]
    q_ref[...] = jnp.dot(hb, wq_ref[...], preferred_element_type=jnp.float32)
    k_ref[...] = jnp.dot(hb, wk_ref[...], preferred_element_type=jnp.float32)
    v_ref[...] = jnp.dot(hb, wv_ref[...], preferred_element_type=jnp.float32)


def _tc_qkv(h, Wq, Wk, Wv):
    return pl.pallas_call(
        _qkv_body,
        grid=(_GRID,),
        in_specs=[
            pl.BlockSpec((_ROWBLK, D), lambda i: (i, 0)),
            pl.BlockSpec((D, D), lambda i: (0, 0)),
            pl.BlockSpec((D, D), lambda i: (0, 0)),
            pl.BlockSpec((D, D), lambda i: (0, 0)),
        ],
        out_specs=[
            pl.BlockSpec((_ROWBLK, D), lambda i: (i, 0)),
            pl.BlockSpec((_ROWBLK, D), lambda i: (i, 0)),
            pl.BlockSpec((_ROWBLK, D), lambda i: (i, 0)),
        ],
        out_shape=[jax.ShapeDtypeStruct((N, D), jnp.float32)] * 3,
    )(h, Wq, Wk, Wv)


def _splat(x, h):
    idx = jnp.full((16, 1), h, jnp.int32)
    return lax.gather(
        x, idx,
        lax.GatherDimensionNumbers(offset_dims=(), collapsed_slice_dims=(0,),
                                   start_index_map=(0,)),
        (1,), mode=lax.GatherScatterMode.PROMISE_IN_BOUNDS)


def _sc_edge_body(qn_hbm, kn_hbm, vn_hbm, eix_hbm, zrows_hbm, out_hbm,
                  eidx, didx2, qb, kb, vb, wrow, wtmp, acc,
                  sem_i0, sem_i1, sem_q0, sem_k0, sem_v0,
                  sem_q1, sem_k1, sem_v1):
    cid = lax.axis_index("c")
    sid = lax.axis_index("s")
    lane = lax.iota(jnp.int32, 16)
    zeros16 = jnp.zeros((16,), jnp.float32)
    cbase = cid * HALF
    isem = (sem_i0, sem_i1)
    gsem = ((sem_q0, sem_k0, sem_v0), (sem_q1, sem_k1, sem_v1))

    row0 = sid * RPT
    pltpu.sync_copy(zrows_hbm, acc.at[pl.ds(row0, RPT)])

    @pl.when(sid == 0)
    def _():
        pltpu.sync_copy(zrows_hbm.at[pl.ds(0, 8)], acc.at[pl.ds(HALF, 8)])

    plsc.subcore_barrier()

    def issue_idx(j, s):
        pltpu.async_copy(eix_hbm.at[sid].at[j], eidx.at[pl.ds(2 * s, 2)],
                         isem[s])

    def wait_idx(s):
        pltpu.make_async_copy(eix_hbm.at[sid].at[0],
                              eidx.at[pl.ds(2 * s, 2)], isem[s]).wait()

    def issue_gathers(s):
        o = s * CHUNK
        pltpu.async_copy(qn_hbm.at[eidx.at[2 * s + 1]],
                         qb.at[pl.ds(o, CHUNK)], gsem[s][0])
        pltpu.async_copy(kn_hbm.at[eidx.at[2 * s]],
                         kb.at[pl.ds(o, CHUNK)], gsem[s][1])
        pltpu.async_copy(vn_hbm.at[eidx.at[2 * s]],
                         vb.at[pl.ds(o, CHUNK)], gsem[s][2])

    def wait_gathers(s):
        o = s * CHUNK
        for buf, sem in zip((qb, kb, vb), gsem[s]):
            pltpu.make_async_copy(qn_hbm.at[pl.ds(0, CHUNK)],
                                  buf.at[pl.ds(o, CHUNK)], sem).wait()

    def compute(s):
        o = s * CHUNK
        for i in range(CHUNK // 16):
            t = eidx[2 * s + 1, pl.ds(i * 16, 16)] - cbase
            ok = (t >= 0) & (t < HALF)
            didx2[pl.ds(i * 16, 16)] = jnp.where(ok, t, HALF)

        def _group(g, c2):
            base = o + g * DH
            rows = lane + base
            ua = [zeros16] * H
            for c in range(DH):
                rot = (lane + c) & (DH - 1)
                for h in range(H):
                    col = rot + (h * DH)
                    qc = plsc.load_gather(qb, [rows, col])
                    kc = plsc.load_gather(kb, [rows, col])
                    ua[h] = ua[h] + qc * kc
            for h in range(H):
                u = jnp.clip(ua[h] * 0.25, -5.0, 5.0)
                wtmp[h, pl.ds(0, 16)] = jnp.exp(u)
            for e in range(DH):
                ec = g * DH + e
                wvec = plsc.load_gather(wtmp, [lane, jnp.full((16,), e, jnp.int32)])
                wrow[ec, pl.ds(D, 16)] = wvec
                for h in range(H):
                    ws = _splat(wvec, h)
                    wrow[ec, pl.ds(h * DH, 16)] = ws * vb[base + e, pl.ds(h * DH, 16)]
            return c2

        lax.fori_loop(0, CHUNK // 16, _group, 0)
        pltpu.sync_copy(wrow, acc.at[didx2], add=True)

    issue_idx(0, 0)
    wait_idx(0)
    issue_gathers(0)
    issue_idx(1, 1)

    def _pair(p, carry):
        wait_gathers(0)
        wait_idx(1)
        issue_gathers(1)

        @pl.when(p < NPAIR - 1)
        def _():
            issue_idx(2 * p + 2, 0)

        compute(0)

        wait_gathers(1)

        @pl.when(p < NPAIR - 1)
        def _():
            wait_idx(0)
            issue_gathers(0)
            issue_idx(2 * p + 3, 1)

        compute(1)
        return carry

    lax.fori_loop(0, NPAIR, _pair, 0)
    plsc.subcore_barrier()

    pltpu.sync_copy(acc.at[pl.ds(row0, RPT)],
                    out_hbm.at[cid].at[pl.ds(row0, RPT)])


@functools.cache
def _sc_edge():
    return pl.kernel(
        _sc_edge_body,
        out_type=jax.ShapeDtypeStruct((NC, HALF, ROWW), jnp.float32),
        mesh=plsc.VectorSubcoreMesh(
            core_axis_name="c", subcore_axis_name="s",
            num_cores=NC, num_subcores=NS),
        compiler_params=pltpu.CompilerParams(
            use_tc_tiling_on_sc=False, needs_layout_passes=False),
        scratch_types=[
            pltpu.VMEM((4, CHUNK), jnp.int32),
            pltpu.VMEM((CHUNK,), jnp.int32),
            pltpu.VMEM((2 * CHUNK, D), jnp.float32),
            pltpu.VMEM((2 * CHUNK, D), jnp.float32),
            pltpu.VMEM((2 * CHUNK, D), jnp.float32),
            pltpu.VMEM((CHUNK, ROWW), jnp.float32),
            pltpu.VMEM((16, 17), jnp.float32),
            pltpu.VMEM_SHARED((ACCR, ROWW), jnp.float32),
        ] + [pltpu.SemaphoreType.DMA] * 8,
    )


def _ln(t, g, b):
    mu = jnp.mean(t, axis=-1, keepdims=True)
    d = t - mu
    var = jnp.mean(d * d, axis=-1, keepdims=True)
    return d / jnp.sqrt(var + 1e-5) * g + b


def _post_body(h_ref, p_ref, wo_ref, l1g_ref, l1b_ref,
               w1_ref, b1_ref, w2_ref, b2_ref, l2g_ref, l2b_ref,
               exp8_ref, o_ref):
    p = p_ref[...]
    num = p[:, :D]
    ssum = p[:, D:D + H]
    inv = 1.0 / (ssum + 1e-30)
    inv128 = jnp.dot(inv, exp8_ref[...], preferred_element_type=jnp.float32)
    agg = num * inv128
    uh = jnp.dot(agg, wo_ref[...], preferred_element_type=jnp.float32)
    h1 = _ln(h_ref[...] + uh, l1g_ref[...], l1b_ref[...])
    z = jnp.maximum(
        jnp.dot(h1, w1_ref[...], preferred_element_type=jnp.float32)
        + b1_ref[...], 0.0)
    ffn = jnp.dot(z, w2_ref[...], preferred_element_type=jnp.float32) + b2_ref[...]
    o_ref[...] = _ln(h1 + ffn, l2g_ref[...], l2b_ref[...])


def _tc_post(h, part, Wo_l, l1g, l1b, W1, b1, W2, b2, l2g, l2b, exp8):
    return pl.pallas_call(
        _post_body,
        grid=(_GRID,),
        in_specs=[
            pl.BlockSpec((_ROWBLK, D), lambda i: (i, 0)),
            pl.BlockSpec((_ROWBLK, ROWW), lambda i: (i, 0)),
            pl.BlockSpec((D, D), lambda i: (0, 0)),
            pl.BlockSpec((1, D), lambda i: (0, 0)),
            pl.BlockSpec((1, D), lambda i: (0, 0)),
            pl.BlockSpec((D, DFF), lambda i: (0, 0)),
            pl.BlockSpec((1, DFF), lambda i: (0, 0)),
            pl.BlockSpec((DFF, D), lambda i: (0, 0)),
            pl.BlockSpec((1, D), lambda i: (0, 0)),
            pl.BlockSpec((1, D), lambda i: (0, 0)),
            pl.BlockSpec((1, D), lambda i: (0, 0)),
            pl.BlockSpec((H, D), lambda i: (0, 0)),
        ],
        out_specs=pl.BlockSpec((_ROWBLK, D), lambda i: (i, 0)),
        out_shape=jax.ShapeDtypeStruct((N, D), jnp.float32),
    )(h, part, Wo_l, l1g.reshape(1, D), l1b.reshape(1, D),
      W1, b1.reshape(1, DFF), W2, b2.reshape(1, D),
      l2g.reshape(1, D), l2b.reshape(1, D), exp8)


def _dec_body(h_ref, wt_ref, b_ref, o_ref):
    i = pl.program_id(0)
    s = jnp.sum(h_ref[...] * wt_ref[...]) * (1.0 / N)

    @pl.when(i == 0)
    def _():
        o_ref[...] = jnp.zeros_like(o_ref)

    o_ref[...] += s

    @pl.when(i == _GRID - 1)
    def _():
        o_ref[...] += b_ref[...]


def _tc_decode(h, dec_W, dec_b):
    return pl.pallas_call(
        _dec_body,
        grid=(_GRID,),
        in_specs=[
            pl.BlockSpec((_ROWBLK, D), lambda i: (i, 0)),
            pl.BlockSpec((1, D), lambda i: (0, 0)),
            pl.BlockSpec((1, 1), lambda i: (0, 0)),
        ],
        out_specs=pl.BlockSpec((1, 1), lambda i: (0, 0)),
        out_shape=jax.ShapeDtypeStruct((1, 1), jnp.float32),
    )(h, dec_W.reshape(1, D), dec_b.reshape(1, 1))


_EXP8 = np.repeat(np.eye(H, dtype=np.float32), DH, axis=1)


def kernel(x, edge_index, enc_W, enc_b, Wq, Wk, Wv, Wo, ln1_g, ln1_b,
           ffn_W1, ffn_b1, ffn_W2, ffn_b2, ln2_g, ln2_b, dec_W, dec_b):
    src = edge_index[0].reshape(NS, NCHUNK, CHUNK)
    dst = edge_index[1].reshape(NS, NCHUNK, CHUNK)
    eix = jnp.stack([src, dst], axis=2)
    zrows = jnp.zeros((RPT, ROWW), jnp.float32)
    exp8 = jnp.asarray(_EXP8)

    h = _tc_encode(x, enc_W, enc_b)
    for l in range(L):
        qn, kn, vn = _tc_qkv(h, Wq[l], Wk[l], Wv[l])
        part = _sc_edge()(qn, kn, vn, eix, zrows).reshape(NC * HALF, ROWW)
        h = _tc_post(h, part, Wo[l], ln1_g[l], ln1_b[l],
                     ffn_W1[l], ffn_b1[l], ffn_W2[l], ffn_b2[l],
                     ln2_g[l], ln2_b[l], exp8)
    return _tc_decode(h, dec_W, dec_b)


# speedup vs baseline: 30.1553x; 1.6041x over previous
import functools

import jax
import jax.numpy as jnp
import numpy as np
from jax import lax
from jax.experimental import pallas as pl
from jax.experimental.pallas import tpu as pltpu
from jax.experimental.pallas import tpu_sc as plsc

N = 10000
E = 320000
D = 128
H = 8
DH = 16
DFF = 512
L = 2

NC = 2
NS = 16
NW = NC * NS
EPS = E // NS
CHUNK = 80
NCHUNK = EPS // CHUNK
ROWW = 144
HALF = 5120
ACCR = HALF + 8
RPT = HALF // NS
SROWS = 8
NPAIR = NCHUNK // 2

_ROWBLK = 1000
_GRID = N // _ROWBLK


def _enc_body(x_ref, w_ref, b_ref, o_ref):
    o_ref[...] = (
        jnp.dot(x_ref[...], w_ref[...], preferred_element_type=jnp.float32)
        + b_ref[...]
    )


def _tc_encode(x, enc_W, enc_b):
    return pl.pallas_call(
        _enc_body,
        grid=(_GRID,),
        in_specs=[
            pl.BlockSpec((_ROWBLK, D), lambda i: (i, 0)),
            pl.BlockSpec((D, D), lambda i: (0, 0)),
            pl.BlockSpec((1, D), lambda i: (0, 0)),
        ],
        out_specs=pl.BlockSpec((_ROWBLK, D), lambda i: (i, 0)),
        out_shape=jax.ShapeDtypeStruct((N, D), jnp.float32),
    )(x, enc_W, enc_b.reshape(1, D))


def _qkv_body(h_ref, wq_ref, wk_ref, wv_ref, q_ref, k_ref, v_ref):
    hb = h_ref[...]
    q_ref[...] = jnp.dot(hb, wq_ref[...], preferred_element_type=jnp.float32)
    k_ref[...] = jnp.dot(hb, wk_ref[...], preferred_element_type=jnp.float32)
    v_ref[...] = jnp.dot(hb, wv_ref[...], preferred_element_type=jnp.float32)


def _tc_qkv(h, Wq, Wk, Wv):
    return pl.pallas_call(
        _qkv_body,
        grid=(_GRID,),
        in_specs=[
            pl.BlockSpec((_ROWBLK, D), lambda i: (i, 0)),
            pl.BlockSpec((D, D), lambda i: (0, 0)),
            pl.BlockSpec((D, D), lambda i: (0, 0)),
            pl.BlockSpec((D, D), lambda i: (0, 0)),
        ],
        out_specs=[
            pl.BlockSpec((_ROWBLK, D), lambda i: (i, 0)),
            pl.BlockSpec((_ROWBLK, D), lambda i: (i, 0)),
            pl.BlockSpec((_ROWBLK, D), lambda i: (i, 0)),
        ],
        out_shape=[jax.ShapeDtypeStruct((N, D), jnp.float32)] * 3,
    )(h, Wq, Wk, Wv)


def _sc_edge_body(qn_hbm, kn_hbm, vn_hbm, eix_hbm, zrows_hbm, out_hbm,
                  eidx, didx2, qb, kb, vb, wrow, wtmp, acc,
                  sem_i0, sem_i1, sem_q0, sem_k0, sem_v0,
                  sem_q1, sem_k1, sem_v1):
    cid = lax.axis_index("c")
    sid = lax.axis_index("s")
    lane = lax.iota(jnp.int32, 16)
    zeros16 = jnp.zeros((16,), jnp.float32)
    cbase = cid * HALF
    isem = (sem_i0, sem_i1)
    gsem = ((sem_q0, sem_k0, sem_v0), (sem_q1, sem_k1, sem_v1))

    row0 = sid * RPT
    pltpu.sync_copy(zrows_hbm, acc.at[pl.ds(row0, RPT)])

    @pl.when(sid == 0)
    def _():
        pltpu.sync_copy(zrows_hbm.at[pl.ds(0, 8)], acc.at[pl.ds(HALF, 8)])

    plsc.subcore_barrier()

    def issue_idx(j, s):
        pltpu.async_copy(eix_hbm.at[sid].at[j], eidx.at[pl.ds(2 * s, 2)],
                         isem[s])

    def wait_idx(s):
        pltpu.make_async_copy(eix_hbm.at[sid].at[0],
                              eidx.at[pl.ds(2 * s, 2)], isem[s]).wait()

    def issue_gathers(s):
        o = s * CHUNK
        pltpu.async_copy(qn_hbm.at[eidx.at[2 * s + 1]],
                         qb.at[pl.ds(o, CHUNK)], gsem[s][0])
        pltpu.async_copy(kn_hbm.at[eidx.at[2 * s]],
                         kb.at[pl.ds(o, CHUNK)], gsem[s][1])
        pltpu.async_copy(vn_hbm.at[eidx.at[2 * s]],
                         vb.at[pl.ds(o, CHUNK)], gsem[s][2])

    def wait_gathers(s):
        o = s * CHUNK
        for buf, sem in zip((qb, kb, vb), gsem[s]):
            pltpu.make_async_copy(qn_hbm.at[pl.ds(0, CHUNK)],
                                  buf.at[pl.ds(o, CHUNK)], sem).wait()

    def compute(s):
        o = s * CHUNK
        for i in range(CHUNK // 16):
            t = eidx[2 * s + 1, pl.ds(i * 16, 16)] - cbase
            ok = (t >= 0) & (t < HALF)
            didx2[pl.ds(i * 16, 16)] = jnp.where(ok, t, HALF)

        def _group(g, c2):
            base = o + g * DH
            rows = lane + base
            wrows = lane + g * DH
            def _p1c(c, ua):
                rot = (lane + c) & (DH - 1)
                return tuple(
                    ua[h] + plsc.load_gather(qb, [rows, rot + h * DH])
                    * plsc.load_gather(kb, [rows, rot + h * DH])
                    for h in range(H))

            ua = lax.fori_loop(0, DH, _p1c, (zeros16,) * H)
            for h in range(H):
                u = jnp.clip(ua[h] * 0.25, -5.0, 5.0)
                wtmp[h, pl.ds(0, 16)] = jnp.exp(u)
            def _p3h(h, c3):
                wh = wtmp[h, pl.ds(0, 16)]
                for c in range(DH):
                    col = ((lane + c) & (DH - 1)) + h * DH
                    vc = plsc.load_gather(vb, [rows, col])
                    plsc.store_scatter(wrow, [wrows, col], wh * vc)
                return c3

            lax.fori_loop(0, H, _p3h, 0)
            for t in range(H):
                hsel = (lane + t) & (H - 1)
                wv = plsc.load_gather(wtmp, [hsel, lane])
                plsc.store_scatter(wrow, [wrows, hsel + D], wv)
            return c2

        lax.fori_loop(0, CHUNK // 16, _group, 0)
        pltpu.sync_copy(wrow, acc.at[didx2], add=True)

    issue_idx(0, 0)
    wait_idx(0)
    issue_gathers(0)
    issue_idx(1, 1)

    def _pair(p, carry):
        wait_gathers(0)
        wait_idx(1)
        issue_gathers(1)

        @pl.when(p < NPAIR - 1)
        def _():
            issue_idx(2 * p + 2, 0)

        compute(0)

        wait_gathers(1)

        @pl.when(p < NPAIR - 1)
        def _():
            wait_idx(0)
            issue_gathers(0)
            issue_idx(2 * p + 3, 1)

        compute(1)
        return carry

    lax.fori_loop(0, NPAIR, _pair, 0)
    plsc.subcore_barrier()

    pltpu.sync_copy(acc.at[pl.ds(row0, RPT)],
                    out_hbm.at[cid].at[pl.ds(row0, RPT)])


@functools.cache
def _sc_edge():
    return pl.kernel(
        _sc_edge_body,
        out_type=jax.ShapeDtypeStruct((NC, HALF, ROWW), jnp.float32),
        mesh=plsc.VectorSubcoreMesh(
            core_axis_name="c", subcore_axis_name="s",
            num_cores=NC, num_subcores=NS),
        compiler_params=pltpu.CompilerParams(
            use_tc_tiling_on_sc=False, needs_layout_passes=False),
        scratch_types=[
            pltpu.VMEM((4, CHUNK), jnp.int32),
            pltpu.VMEM((CHUNK,), jnp.int32),
            pltpu.VMEM((2 * CHUNK, D), jnp.float32),
            pltpu.VMEM((2 * CHUNK, D), jnp.float32),
            pltpu.VMEM((2 * CHUNK, D), jnp.float32),
            pltpu.VMEM((CHUNK, ROWW), jnp.float32),
            pltpu.VMEM((16, 17), jnp.float32),
            pltpu.VMEM_SHARED((ACCR, ROWW), jnp.float32),
        ] + [pltpu.SemaphoreType.DMA] * 8,
    )


def _ln(t, g, b):
    mu = jnp.mean(t, axis=-1, keepdims=True)
    d = t - mu
    var = jnp.mean(d * d, axis=-1, keepdims=True)
    return d / jnp.sqrt(var + 1e-5) * g + b


def _post_body(h_ref, p_ref, wo_ref, l1g_ref, l1b_ref,
               w1_ref, b1_ref, w2_ref, b2_ref, l2g_ref, l2b_ref,
               exp8_ref, o_ref):
    p = p_ref[...]
    num = p[:, :D]
    ssum = p[:, D:D + H]
    inv = 1.0 / (ssum + 1e-30)
    inv128 = jnp.dot(inv, exp8_ref[...], preferred_element_type=jnp.float32)
    agg = num * inv128
    uh = jnp.dot(agg, wo_ref[...], preferred_element_type=jnp.float32)
    h1 = _ln(h_ref[...] + uh, l1g_ref[...], l1b_ref[...])
    z = jnp.maximum(
        jnp.dot(h1, w1_ref[...], preferred_element_type=jnp.float32)
        + b1_ref[...], 0.0)
    ffn = jnp.dot(z, w2_ref[...], preferred_element_type=jnp.float32) + b2_ref[...]
    o_ref[...] = _ln(h1 + ffn, l2g_ref[...], l2b_ref[...])


def _tc_post(h, part, Wo_l, l1g, l1b, W1, b1, W2, b2, l2g, l2b, exp8):
    return pl.pallas_call(
        _post_body,
        grid=(_GRID,),
        in_specs=[
            pl.BlockSpec((_ROWBLK, D), lambda i: (i, 0)),
            pl.BlockSpec((_ROWBLK, ROWW), lambda i: (i, 0)),
            pl.BlockSpec((D, D), lambda i: (0, 0)),
            pl.BlockSpec((1, D), lambda i: (0, 0)),
            pl.BlockSpec((1, D), lambda i: (0, 0)),
            pl.BlockSpec((D, DFF), lambda i: (0, 0)),
            pl.BlockSpec((1, DFF), lambda i: (0, 0)),
            pl.BlockSpec((DFF, D), lambda i: (0, 0)),
            pl.BlockSpec((1, D), lambda i: (0, 0)),
            pl.BlockSpec((1, D), lambda i: (0, 0)),
            pl.BlockSpec((1, D), lambda i: (0, 0)),
            pl.BlockSpec((H, D), lambda i: (0, 0)),
        ],
        out_specs=pl.BlockSpec((_ROWBLK, D), lambda i: (i, 0)),
        out_shape=jax.ShapeDtypeStruct((N, D), jnp.float32),
    )(h, part, Wo_l, l1g.reshape(1, D), l1b.reshape(1, D),
      W1, b1.reshape(1, DFF), W2, b2.reshape(1, D),
      l2g.reshape(1, D), l2b.reshape(1, D), exp8)


def _dec_body(h_ref, wt_ref, b_ref, o_ref):
    i = pl.program_id(0)
    s = jnp.sum(h_ref[...] * wt_ref[...]) * (1.0 / N)

    @pl.when(i == 0)
    def _():
        o_ref[...] = jnp.zeros_like(o_ref)

    o_ref[...] += s

    @pl.when(i == _GRID - 1)
    def _():
        o_ref[...] += b_ref[...]


def _tc_decode(h, dec_W, dec_b):
    return pl.pallas_call(
        _dec_body,
        grid=(_GRID,),
        in_specs=[
            pl.BlockSpec((_ROWBLK, D), lambda i: (i, 0)),
            pl.BlockSpec((1, D), lambda i: (0, 0)),
            pl.BlockSpec((1, 1), lambda i: (0, 0)),
        ],
        out_specs=pl.BlockSpec((1, 1), lambda i: (0, 0)),
        out_shape=jax.ShapeDtypeStruct((1, 1), jnp.float32),
    )(h, dec_W.reshape(1, D), dec_b.reshape(1, 1))


_EXP8 = np.repeat(np.eye(H, dtype=np.float32), DH, axis=1)


def kernel(x, edge_index, enc_W, enc_b, Wq, Wk, Wv, Wo, ln1_g, ln1_b,
           ffn_W1, ffn_b1, ffn_W2, ffn_b2, ln2_g, ln2_b, dec_W, dec_b):
    src = edge_index[0].reshape(NS, NCHUNK, CHUNK)
    dst = edge_index[1].reshape(NS, NCHUNK, CHUNK)
    eix = jnp.stack([src, dst], axis=2)
    zrows = jnp.zeros((RPT, ROWW), jnp.float32)
    exp8 = jnp.asarray(_EXP8)

    h = _tc_encode(x, enc_W, enc_b)
    for l in range(L):
        qn, kn, vn = _tc_qkv(h, Wq[l], Wk[l], Wv[l])
        part = _sc_edge()(qn, kn, vn, eix, zrows).reshape(NC * HALF, ROWW)
        h = _tc_post(h, part, Wo[l], ln1_g[l], ln1_b[l],
                     ffn_W1[l], ffn_b1[l], ffn_W2[l], ffn_b2[l],
                     ln2_g[l], ln2_b[l], exp8)
    return _tc_decode(h, dec_W, dec_b)


# speedup vs baseline: 31.4309x; 1.0423x over previous
import functools

import jax
import jax.numpy as jnp
import numpy as np
from jax import lax
from jax.experimental import pallas as pl
from jax.experimental.pallas import tpu as pltpu
from jax.experimental.pallas import tpu_sc as plsc

N = 10000
E = 320000
D = 128
H = 8
DH = 16
DFF = 512
L = 2

NC = 2
NS = 16
NW = NC * NS
EPS = E // NS
CHUNK = 80
NCHUNK = EPS // CHUNK
ROWW = 144
HALF = 5120
ACCR = HALF + 8
RPT = HALF // NS
SROWS = 8
NPAIR = NCHUNK // 2

_ROWBLK = 1000
_GRID = N // _ROWBLK


def _enc_body(x_ref, w_ref, b_ref, o_ref):
    o_ref[...] = (
        jnp.dot(x_ref[...], w_ref[...], preferred_element_type=jnp.float32)
        + b_ref[...]
    )


def _tc_encode(x, enc_W, enc_b):
    return pl.pallas_call(
        _enc_body,
        grid=(_GRID,),
        in_specs=[
            pl.BlockSpec((_ROWBLK, D), lambda i: (i, 0)),
            pl.BlockSpec((D, D), lambda i: (0, 0)),
            pl.BlockSpec((1, D), lambda i: (0, 0)),
        ],
        out_specs=pl.BlockSpec((_ROWBLK, D), lambda i: (i, 0)),
        out_shape=jax.ShapeDtypeStruct((N, D), jnp.float32),
    )(x, enc_W, enc_b.reshape(1, D))


def _qkv_body(h_ref, wqe_ref, wqo_ref, wke_ref, wko_ref, wv_ref,
              qp_ref, kp_ref, v_ref):
    hb = h_ref[...]

    def pack2(we_ref, wo_ref):
        e = jnp.dot(hb, we_ref[...],
                    preferred_element_type=jnp.float32).astype(jnp.bfloat16)
        o = jnp.dot(hb, wo_ref[...],
                    preferred_element_type=jnp.float32).astype(jnp.bfloat16)
        eu = lax.bitcast_convert_type(e, jnp.uint16).astype(jnp.uint32)
        ou = lax.bitcast_convert_type(o, jnp.uint16).astype(jnp.uint32)
        return lax.bitcast_convert_type(eu | (ou << 16), jnp.int32)

    qp_ref[...] = pack2(wqe_ref, wqo_ref)
    kp_ref[...] = pack2(wke_ref, wko_ref)
    v_ref[...] = jnp.dot(hb, wv_ref[...], preferred_element_type=jnp.float32)


def _tc_qkv(h, Wqe, Wqo, Wke, Wko, Wv):
    return pl.pallas_call(
        _qkv_body,
        grid=(_GRID,),
        in_specs=[
            pl.BlockSpec((_ROWBLK, D), lambda i: (i, 0)),
            pl.BlockSpec((D, D // 2), lambda i: (0, 0)),
            pl.BlockSpec((D, D // 2), lambda i: (0, 0)),
            pl.BlockSpec((D, D // 2), lambda i: (0, 0)),
            pl.BlockSpec((D, D // 2), lambda i: (0, 0)),
            pl.BlockSpec((D, D), lambda i: (0, 0)),
        ],
        out_specs=[
            pl.BlockSpec((_ROWBLK, D // 2), lambda i: (i, 0)),
            pl.BlockSpec((_ROWBLK, D // 2), lambda i: (i, 0)),
            pl.BlockSpec((_ROWBLK, D), lambda i: (i, 0)),
        ],
        out_shape=[jax.ShapeDtypeStruct((N, D // 2), jnp.int32),
                   jax.ShapeDtypeStruct((N, D // 2), jnp.int32),
                   jax.ShapeDtypeStruct((N, D), jnp.float32)],
    )(h, Wqe, Wqo, Wke, Wko, Wv)


def _sc_edge_body(qn_hbm, kn_hbm, vn_hbm, eix_hbm, zrows_hbm, out_hbm,
                  eidx, didx2, qb, kb, vb, wrow, wtmp, acc,
                  sem_i0, sem_i1, sem_q0, sem_k0, sem_v0,
                  sem_q1, sem_k1, sem_v1):
    cid = lax.axis_index("c")
    sid = lax.axis_index("s")
    lane = lax.iota(jnp.int32, 16)
    zeros16 = jnp.zeros((16,), jnp.float32)
    cbase = cid * HALF
    isem = (sem_i0, sem_i1)
    gsem = ((sem_q0, sem_k0, sem_v0), (sem_q1, sem_k1, sem_v1))

    row0 = sid * RPT
    pltpu.sync_copy(zrows_hbm, acc.at[pl.ds(row0, RPT)])

    @pl.when(sid == 0)
    def _():
        pltpu.sync_copy(zrows_hbm.at[pl.ds(0, 8)], acc.at[pl.ds(HALF, 8)])

    plsc.subcore_barrier()

    def issue_idx(j, s):
        pltpu.async_copy(eix_hbm.at[sid].at[j], eidx.at[pl.ds(2 * s, 2)],
                         isem[s])

    def wait_idx(s):
        pltpu.make_async_copy(eix_hbm.at[sid].at[0],
                              eidx.at[pl.ds(2 * s, 2)], isem[s]).wait()

    def issue_gathers(s):
        o = s * CHUNK
        pltpu.async_copy(qn_hbm.at[eidx.at[2 * s + 1]],
                         qb.at[pl.ds(o, CHUNK)], gsem[s][0])
        pltpu.async_copy(kn_hbm.at[eidx.at[2 * s]],
                         kb.at[pl.ds(o, CHUNK)], gsem[s][1])
        pltpu.async_copy(vn_hbm.at[eidx.at[2 * s]],
                         vb.at[pl.ds(o, CHUNK)], gsem[s][2])

    def wait_gathers(s):
        o = s * CHUNK
        for src, buf, sem in zip((qn_hbm, kn_hbm, vn_hbm), (qb, kb, vb),
                                 gsem[s]):
            pltpu.make_async_copy(src.at[pl.ds(0, CHUNK)],
                                  buf.at[pl.ds(o, CHUNK)], sem).wait()

    def compute(s):
        o = s * CHUNK
        for i in range(CHUNK // 16):
            t = eidx[2 * s + 1, pl.ds(i * 16, 16)] - cbase
            ok = (t >= 0) & (t < HALF)
            didx2[pl.ds(i * 16, 16)] = jnp.where(ok, t, HALF)

        def _group(g, c2):
            base = o + g * DH
            rows = lane + base
            wrows = lane + g * DH
            def _p1c(c, ua):
                rot = (lane + c) & (DH // 2 - 1)
                new = []
                for h in range(H):
                    col = rot + h * (DH // 2)
                    qw = plsc.bitcast(plsc.load_gather(qb, [rows, col]),
                                      jnp.bfloat16)
                    kw = plsc.bitcast(plsc.load_gather(kb, [rows, col]),
                                      jnp.bfloat16)
                    qe, qo = plsc.unpack(qw, format=plsc.PackFormat.INTERLEAVED)
                    ke, ko = plsc.unpack(kw, format=plsc.PackFormat.INTERLEAVED)
                    new.append(ua[h] + (qe * ke + qo * ko))
                return tuple(new)

            ua = lax.fori_loop(0, DH // 2, _p1c, (zeros16,) * H)
            for h in range(H):
                u = jnp.clip(ua[h], -5.0, 5.0)
                wtmp[h, pl.ds(0, 16)] = jnp.exp(u)
            def _p3h(h, c3):
                wh = wtmp[h, pl.ds(0, 16)]
                for c in range(DH):
                    col = ((lane + c) & (DH - 1)) + h * DH
                    vc = plsc.load_gather(vb, [rows, col])
                    plsc.store_scatter(wrow, [wrows, col], wh * vc)
                return c3

            lax.fori_loop(0, H, _p3h, 0)
            for t in range(H):
                hsel = (lane + t) & (H - 1)
                wv = plsc.load_gather(wtmp, [hsel, lane])
                plsc.store_scatter(wrow, [wrows, hsel + D], wv)
            return c2

        lax.fori_loop(0, CHUNK // 16, _group, 0)
        pltpu.sync_copy(wrow, acc.at[didx2], add=True)

    issue_idx(0, 0)
    wait_idx(0)
    issue_gathers(0)
    issue_idx(1, 1)

    def _pair(p, carry):
        wait_gathers(0)
        wait_idx(1)
        issue_gathers(1)

        @pl.when(p < NPAIR - 1)
        def _():
            issue_idx(2 * p + 2, 0)

        compute(0)

        wait_gathers(1)

        @pl.when(p < NPAIR - 1)
        def _():
            wait_idx(0)
            issue_gathers(0)
            issue_idx(2 * p + 3, 1)

        compute(1)
        return carry

    lax.fori_loop(0, NPAIR, _pair, 0)
    plsc.subcore_barrier()

    pltpu.sync_copy(acc.at[pl.ds(row0, RPT)],
                    out_hbm.at[cid].at[pl.ds(row0, RPT)])


@functools.cache
def _sc_edge():
    return pl.kernel(
        _sc_edge_body,
        out_type=jax.ShapeDtypeStruct((NC, HALF, ROWW), jnp.float32),
        mesh=plsc.VectorSubcoreMesh(
            core_axis_name="c", subcore_axis_name="s",
            num_cores=NC, num_subcores=NS),
        compiler_params=pltpu.CompilerParams(
            use_tc_tiling_on_sc=False, needs_layout_passes=False),
        scratch_types=[
            pltpu.VMEM((4, CHUNK), jnp.int32),
            pltpu.VMEM((CHUNK,), jnp.int32),
            pltpu.VMEM((2 * CHUNK, D // 2), jnp.int32),
            pltpu.VMEM((2 * CHUNK, D // 2), jnp.int32),
            pltpu.VMEM((2 * CHUNK, D), jnp.float32),
            pltpu.VMEM((CHUNK, ROWW), jnp.float32),
            pltpu.VMEM((16, 17), jnp.float32),
            pltpu.VMEM_SHARED((ACCR, ROWW), jnp.float32),
        ] + [pltpu.SemaphoreType.DMA] * 8,
    )


def _ln(t, g, b):
    mu = jnp.mean(t, axis=-1, keepdims=True)
    d = t - mu
    var = jnp.mean(d * d, axis=-1, keepdims=True)
    return d / jnp.sqrt(var + 1e-5) * g + b


def _post_body(h_ref, p_ref, wo_ref, l1g_ref, l1b_ref,
               w1_ref, b1_ref, w2_ref, b2_ref, l2g_ref, l2b_ref,
               exp8_ref, o_ref):
    p = p_ref[...]
    num = p[:, :D]
    ssum = p[:, D:D + H]
    inv = 1.0 / (ssum + 1e-30)
    inv128 = jnp.dot(inv, exp8_ref[...], preferred_element_type=jnp.float32)
    agg = num * inv128
    uh = jnp.dot(agg, wo_ref[...], preferred_element_type=jnp.float32)
    h1 = _ln(h_ref[...] + uh, l1g_ref[...], l1b_ref[...])
    z = jnp.maximum(
        jnp.dot(h1, w1_ref[...], preferred_element_type=jnp.float32)
        + b1_ref[...], 0.0)
    ffn = jnp.dot(z, w2_ref[...], preferred_element_type=jnp.float32) + b2_ref[...]
    o_ref[...] = _ln(h1 + ffn, l2g_ref[...], l2b_ref[...])


def _tc_post(h, part, Wo_l, l1g, l1b, W1, b1, W2, b2, l2g, l2b, exp8):
    return pl.pallas_call(
        _post_body,
        grid=(_GRID,),
        in_specs=[
            pl.BlockSpec((_ROWBLK, D), lambda i: (i, 0)),
            pl.BlockSpec((_ROWBLK, ROWW), lambda i: (i, 0)),
            pl.BlockSpec((D, D), lambda i: (0, 0)),
            pl.BlockSpec((1, D), lambda i: (0, 0)),
            pl.BlockSpec((1, D), lambda i: (0, 0)),
            pl.BlockSpec((D, DFF), lambda i: (0, 0)),
            pl.BlockSpec((1, DFF), lambda i: (0, 0)),
            pl.BlockSpec((DFF, D), lambda i: (0, 0)),
            pl.BlockSpec((1, D), lambda i: (0, 0)),
            pl.BlockSpec((1, D), lambda i: (0, 0)),
            pl.BlockSpec((1, D), lambda i: (0, 0)),
            pl.BlockSpec((H, D), lambda i: (0, 0)),
        ],
        out_specs=pl.BlockSpec((_ROWBLK, D), lambda i: (i, 0)),
        out_shape=jax.ShapeDtypeStruct((N, D), jnp.float32),
    )(h, part, Wo_l, l1g.reshape(1, D), l1b.reshape(1, D),
      W1, b1.reshape(1, DFF), W2, b2.reshape(1, D),
      l2g.reshape(1, D), l2b.reshape(1, D), exp8)


def _dec_body(h_ref, wt_ref, b_ref, o_ref):
    i = pl.program_id(0)
    s = jnp.sum(h_ref[...] * wt_ref[...]) * (1.0 / N)

    @pl.when(i == 0)
    def _():
        o_ref[...] = jnp.zeros_like(o_ref)

    o_ref[...] += s

    @pl.when(i == _GRID - 1)
    def _():
        o_ref[...] += b_ref[...]


def _tc_decode(h, dec_W, dec_b):
    return pl.pallas_call(
        _dec_body,
        grid=(_GRID,),
        in_specs=[
            pl.BlockSpec((_ROWBLK, D), lambda i: (i, 0)),
            pl.BlockSpec((1, D), lambda i: (0, 0)),
            pl.BlockSpec((1, 1), lambda i: (0, 0)),
        ],
        out_specs=pl.BlockSpec((1, 1), lambda i: (0, 0)),
        out_shape=jax.ShapeDtypeStruct((1, 1), jnp.float32),
    )(h, dec_W.reshape(1, D), dec_b.reshape(1, 1))


_EXP8 = np.repeat(np.eye(H, dtype=np.float32), DH, axis=1)


def kernel(x, edge_index, enc_W, enc_b, Wq, Wk, Wv, Wo, ln1_g, ln1_b,
           ffn_W1, ffn_b1, ffn_W2, ffn_b2, ln2_g, ln2_b, dec_W, dec_b):
    src = edge_index[0].reshape(NS, NCHUNK, CHUNK)
    dst = edge_index[1].reshape(NS, NCHUNK, CHUNK)
    eix = jnp.stack([src, dst], axis=2)
    zrows = jnp.zeros((RPT, ROWW), jnp.float32)
    exp8 = jnp.asarray(_EXP8)

    h = _tc_encode(x, enc_W, enc_b)
    qscale = 1.0 / np.sqrt(DH)
    for l in range(L):
        qp, kp, vn = _tc_qkv(h, Wq[l][:, 0::2] * qscale, Wq[l][:, 1::2] * qscale,
                             Wk[l][:, 0::2], Wk[l][:, 1::2], Wv[l])
        part = _sc_edge()(qp, kp, vn, eix, zrows).reshape(NC * HALF, ROWW)
        h = _tc_post(h, part, Wo[l], ln1_g[l], ln1_b[l],
                     ffn_W1[l], ffn_b1[l], ffn_W2[l], ffn_b2[l],
                     ln2_g[l], ln2_b[l], exp8)
    return _tc_decode(h, dec_W, dec_b)
